# baseline (device time: 9765 ns/iter reference)
import jax
import jax.numpy as jnp
from jax import lax
from jax.experimental import pallas as pl
from jax.experimental.pallas import tpu as pltpu

N_GLOBAL = 1024
EPS = 1e-5
C = 8
H = C // 2


def kernel(x, gamma):
    m, n = x.shape
    rows = m // C
    x3 = x.reshape(C, rows, n)
    gamma2 = gamma.reshape(1, n)

    def body(
        x_hbm, g_ref, o_hbm,
        x_vmem, o_vmem, col_buf, send_pack, recv_pack,
        in_sems, out_sems, send_sems, recv_sems,
    ):
        my_x = lax.axis_index("x")
        my_y = lax.axis_index("y")
        nbr = (my_x, 1 - my_y)

        barrier_sem = pltpu.get_barrier_semaphore()
        pl.semaphore_signal(
            barrier_sem, inc=1, device_id=nbr,
            device_id_type=pl.DeviceIdType.MESH,
        )

        in_copies = []
        for c in range(C):
            cp = pltpu.make_async_copy(
                x_hbm.at[c], x_vmem.at[c], in_sems.at[c]
            )
            cp.start()
            in_copies.append(cp)

        def make_half_rdma(h):
            sl = pl.ds(h * H, H)
            return pltpu.make_async_remote_copy(
                src_ref=send_pack.at[sl, :],
                dst_ref=recv_pack.at[sl, :],
                send_sem=send_sems.at[h],
                recv_sem=recv_sems.at[h],
                device_id=nbr,
                device_id_type=pl.DeviceIdType.MESH,
            )

        rdmas = []
        for c in range(C):
            in_copies[c].wait()
            xc = x_vmem[c]
            col_buf[:, pl.ds(c, 1)] = jnp.sum(xc * xc, axis=1, keepdims=True)
            if c == H - 1:
                pl.semaphore_wait(barrier_sem, 1)
            if c % H == H - 1:
                h = c // H
                sl = pl.ds(h * H, H)
                send_pack[sl, :] = jnp.swapaxes(
                    col_buf[:, pl.ds(h * H, H)], 0, 1
                )
                rdma = make_half_rdma(h)
                rdma.start()
                rdmas.append(rdma)

        out_copies = []
        for h in range(2):
            sl = pl.ds(h * H, H)
            rdmas[h].wait_recv()
            total = send_pack[sl, :] + recv_pack[sl, :]
            inv_rms = lax.rsqrt(total * (1.0 / N_GLOBAL) + EPS)
            inv_cols = jnp.swapaxes(inv_rms, 0, 1)
            for k in range(H):
                c = h * H + k
                inv_col = inv_cols[:, k:k + 1]
                o_vmem[c] = (
                    x_vmem[c] * g_ref[:, :] * inv_col
                ).astype(jnp.bfloat16)
                cp = pltpu.make_async_copy(
                    o_vmem.at[c], o_hbm.at[c], out_sems.at[c]
                )
                cp.start()
                out_copies.append(cp)

        for h in range(2):
            rdmas[h].wait_send()
        for c in range(C):
            out_copies[c].wait()

    out3 = pl.pallas_call(
        body,
        out_shape=jax.ShapeDtypeStruct((C, rows, n), jnp.bfloat16),
        in_specs=[
            pl.BlockSpec(memory_space=pl.ANY),
            pl.BlockSpec(memory_space=pltpu.VMEM),
        ],
        out_specs=pl.BlockSpec(memory_space=pl.ANY),
        scratch_shapes=[
            pltpu.VMEM((C, rows, n), jnp.float32),
            pltpu.VMEM((C, rows, n), jnp.bfloat16),
            pltpu.VMEM((rows, C), jnp.float32),
            pltpu.VMEM((C, rows), jnp.float32),
            pltpu.VMEM((C, rows), jnp.float32),
            pltpu.SemaphoreType.DMA((C,)),
            pltpu.SemaphoreType.DMA((C,)),
            pltpu.SemaphoreType.DMA((2,)),
            pltpu.SemaphoreType.DMA((2,)),
        ],
        compiler_params=pltpu.CompilerParams(collective_id=0),
    )(x3, gamma2)
    return out3.reshape(m, n)


# device time: 8227 ns/iter; 1.1869x vs baseline; 1.1869x over previous
import jax
import jax.numpy as jnp
from jax import lax
from jax.experimental import pallas as pl
from jax.experimental.pallas import tpu as pltpu

N_GLOBAL = 1024
EPS = 1e-5
C = 4


def kernel(x, gamma):
    m, n = x.shape
    rows = m // C
    x3 = x.reshape(C, rows, n)
    gamma2 = gamma.reshape(1, n)

    def body(
        x_hbm, g_ref, o_hbm,
        x_vmem, o_vmem, send_pack, recv_pack,
        in_sems, out_sems, send_sems, recv_sems,
    ):
        my_x = lax.axis_index("x")
        my_y = lax.axis_index("y")
        nbr = (my_x, 1 - my_y)

        barrier_sem = pltpu.get_barrier_semaphore()
        pl.semaphore_signal(
            barrier_sem, inc=1, device_id=nbr,
            device_id_type=pl.DeviceIdType.MESH,
        )

        in_copies = []
        for c in range(C):
            cp = pltpu.make_async_copy(
                x_hbm.at[c], x_vmem.at[c], in_sems.at[c]
            )
            cp.start()
            in_copies.append(cp)

        pl.semaphore_wait(barrier_sem, 1)

        rdmas = []
        for c in range(C):
            in_copies[c].wait()
            xc = x_vmem[c]
            send_pack[c, :] = jnp.sum(xc * xc, axis=1)
            rdma = pltpu.make_async_remote_copy(
                src_ref=send_pack.at[pl.ds(c, 1), :],
                dst_ref=recv_pack.at[pl.ds(c, 1), :],
                send_sem=send_sems.at[c],
                recv_sem=recv_sems.at[c],
                device_id=nbr,
                device_id_type=pl.DeviceIdType.MESH,
            )
            rdma.start()
            rdmas.append(rdma)

        out_copies = []
        for c in range(C):
            rdmas[c].wait_recv()
            total = send_pack[pl.ds(c, 1), :] + recv_pack[pl.ds(c, 1), :]
            inv_rms = lax.rsqrt(total * (1.0 / N_GLOBAL) + EPS)
            inv_col = jnp.reshape(inv_rms, (rows, 1))
            o_vmem[c] = (x_vmem[c] * g_ref[:, :] * inv_col).astype(jnp.bfloat16)
            cp = pltpu.make_async_copy(
                o_vmem.at[c], o_hbm.at[c], out_sems.at[c]
            )
            cp.start()
            out_copies.append(cp)

        for c in range(C):
            rdmas[c].wait_send()
            out_copies[c].wait()

    out3 = pl.pallas_call(
        body,
        out_shape=jax.ShapeDtypeStruct((C, rows, n), jnp.bfloat16),
        in_specs=[
            pl.BlockSpec(memory_space=pl.ANY),
            pl.BlockSpec(memory_space=pltpu.VMEM),
        ],
        out_specs=pl.BlockSpec(memory_space=pl.ANY),
        scratch_shapes=[
            pltpu.VMEM((C, rows, n), jnp.float32),
            pltpu.VMEM((C, rows, n), jnp.bfloat16),
            pltpu.VMEM((C, rows), jnp.float32),
            pltpu.VMEM((C, rows), jnp.float32),
            pltpu.SemaphoreType.DMA((C,)),
            pltpu.SemaphoreType.DMA((C,)),
            pltpu.SemaphoreType.DMA((C,)),
            pltpu.SemaphoreType.DMA((C,)),
        ],
        compiler_params=pltpu.CompilerParams(collective_id=0),
    )(x3, gamma2)
    return out3.reshape(m, n)


# device time: 7972 ns/iter; 1.2249x vs baseline; 1.0320x over previous
import jax
import jax.numpy as jnp
from jax import lax
from jax.experimental import pallas as pl
from jax.experimental.pallas import tpu as pltpu

N_GLOBAL = 1024
EPS = 1e-5
C = 2


def kernel(x, gamma):
    m, n = x.shape
    rows = m // C
    x3 = x.reshape(C, rows, n)
    gamma2 = gamma.reshape(1, n)

    def body(
        x_hbm, g_ref, o_hbm,
        x_vmem, o_vmem, send_pack, recv_pack,
        in_sems, out_sems, send_sems, recv_sems,
    ):
        my_x = lax.axis_index("x")
        my_y = lax.axis_index("y")
        nbr = (my_x, 1 - my_y)

        barrier_sem = pltpu.get_barrier_semaphore()
        pl.semaphore_signal(
            barrier_sem, inc=1, device_id=nbr,
            device_id_type=pl.DeviceIdType.MESH,
        )

        in_copies = []
        for c in range(C):
            cp = pltpu.make_async_copy(
                x_hbm.at[c], x_vmem.at[c], in_sems.at[c]
            )
            cp.start()
            in_copies.append(cp)

        pl.semaphore_wait(barrier_sem, 1)

        rdmas = []
        for c in range(C):
            in_copies[c].wait()
            xc = x_vmem[c]
            send_pack[c, :] = jnp.sum(xc * xc, axis=1)
            rdma = pltpu.make_async_remote_copy(
                src_ref=send_pack.at[pl.ds(c, 1), :],
                dst_ref=recv_pack.at[pl.ds(c, 1), :],
                send_sem=send_sems.at[c],
                recv_sem=recv_sems.at[c],
                device_id=nbr,
                device_id_type=pl.DeviceIdType.MESH,
            )
            rdma.start()
            rdmas.append(rdma)

        out_copies = []
        for c in range(C):
            rdmas[c].wait_recv()
            total = send_pack[pl.ds(c, 1), :] + recv_pack[pl.ds(c, 1), :]
            inv_rms = lax.rsqrt(total * (1.0 / N_GLOBAL) + EPS)
            inv_col = jnp.reshape(inv_rms, (rows, 1))
            o_vmem[c] = (x_vmem[c] * g_ref[:, :] * inv_col).astype(jnp.bfloat16)
            cp = pltpu.make_async_copy(
                o_vmem.at[c], o_hbm.at[c], out_sems.at[c]
            )
            cp.start()
            out_copies.append(cp)

        for c in range(C):
            rdmas[c].wait_send()
            out_copies[c].wait()

    out3 = pl.pallas_call(
        body,
        out_shape=jax.ShapeDtypeStruct((C, rows, n), jnp.bfloat16),
        in_specs=[
            pl.BlockSpec(memory_space=pl.ANY),
            pl.BlockSpec(memory_space=pltpu.VMEM),
        ],
        out_specs=pl.BlockSpec(memory_space=pl.ANY),
        scratch_shapes=[
            pltpu.VMEM((C, rows, n), jnp.float32),
            pltpu.VMEM((C, rows, n), jnp.bfloat16),
            pltpu.VMEM((C, rows), jnp.float32),
            pltpu.VMEM((C, rows), jnp.float32),
            pltpu.SemaphoreType.DMA((C,)),
            pltpu.SemaphoreType.DMA((C,)),
            pltpu.SemaphoreType.DMA((C,)),
            pltpu.SemaphoreType.DMA((C,)),
        ],
        compiler_params=pltpu.CompilerParams(collective_id=0),
    )(x3, gamma2)
    return out3.reshape(m, n)


# device time: 7960 ns/iter; 1.2268x vs baseline; 1.0015x over previous
import jax
import jax.numpy as jnp
from jax import lax
from jax.experimental import pallas as pl
from jax.experimental.pallas import tpu as pltpu

N_GLOBAL = 1024
EPS = 1e-5
C = 1


def kernel(x, gamma):
    m, n = x.shape
    rows = m // C
    x3 = x.reshape(C, rows, n)
    gamma2 = gamma.reshape(1, n)

    def body(
        x_hbm, g_ref, o_hbm,
        x_vmem, o_vmem, send_pack, recv_pack,
        in_sems, out_sems, send_sems, recv_sems,
    ):
        my_x = lax.axis_index("x")
        my_y = lax.axis_index("y")
        nbr = (my_x, 1 - my_y)

        barrier_sem = pltpu.get_barrier_semaphore()
        pl.semaphore_signal(
            barrier_sem, inc=1, device_id=nbr,
            device_id_type=pl.DeviceIdType.MESH,
        )

        in_copies = []
        for c in range(C):
            cp = pltpu.make_async_copy(
                x_hbm.at[c], x_vmem.at[c], in_sems.at[c]
            )
            cp.start()
            in_copies.append(cp)

        pl.semaphore_wait(barrier_sem, 1)

        rdmas = []
        for c in range(C):
            in_copies[c].wait()
            xc = x_vmem[c]
            send_pack[c, :] = jnp.sum(xc * xc, axis=1)
            rdma = pltpu.make_async_remote_copy(
                src_ref=send_pack.at[pl.ds(c, 1), :],
                dst_ref=recv_pack.at[pl.ds(c, 1), :],
                send_sem=send_sems.at[c],
                recv_sem=recv_sems.at[c],
                device_id=nbr,
                device_id_type=pl.DeviceIdType.MESH,
            )
            rdma.start()
            rdmas.append(rdma)

        out_copies = []
        for c in range(C):
            rdmas[c].wait_recv()
            total = send_pack[pl.ds(c, 1), :] + recv_pack[pl.ds(c, 1), :]
            inv_rms = lax.rsqrt(total * (1.0 / N_GLOBAL) + EPS)
            inv_col = jnp.reshape(inv_rms, (rows, 1))
            o_vmem[c] = (x_vmem[c] * g_ref[:, :] * inv_col).astype(jnp.bfloat16)
            cp = pltpu.make_async_copy(
                o_vmem.at[c], o_hbm.at[c], out_sems.at[c]
            )
            cp.start()
            out_copies.append(cp)

        for c in range(C):
            rdmas[c].wait_send()
            out_copies[c].wait()

    out3 = pl.pallas_call(
        body,
        out_shape=jax.ShapeDtypeStruct((C, rows, n), jnp.bfloat16),
        in_specs=[
            pl.BlockSpec(memory_space=pl.ANY),
            pl.BlockSpec(memory_space=pltpu.VMEM),
        ],
        out_specs=pl.BlockSpec(memory_space=pl.ANY),
        scratch_shapes=[
            pltpu.VMEM((C, rows, n), jnp.float32),
            pltpu.VMEM((C, rows, n), jnp.bfloat16),
            pltpu.VMEM((C, rows), jnp.float32),
            pltpu.VMEM((C, rows), jnp.float32),
            pltpu.SemaphoreType.DMA((C,)),
            pltpu.SemaphoreType.DMA((C,)),
            pltpu.SemaphoreType.DMA((C,)),
            pltpu.SemaphoreType.DMA((C,)),
        ],
        compiler_params=pltpu.CompilerParams(collective_id=0),
    )(x3, gamma2)
    return out3.reshape(m, n)


# device time: 7937 ns/iter; 1.2303x vs baseline; 1.0029x over previous
import jax
import jax.numpy as jnp
from jax import lax
from jax.experimental import pallas as pl
from jax.experimental.pallas import tpu as pltpu

N_GLOBAL = 1024
EPS = 1e-5
C = 2


def kernel(x, gamma):
    m, n = x.shape
    rows = m // C
    x3 = x.reshape(C, rows, n)
    gamma2 = gamma.reshape(1, n)

    def body(
        x_hbm, g_ref, o_hbm,
        x_vmem, o_vmem, send_pack, recv_pack,
        in_sems, out_sems, send_sems, recv_sems,
    ):
        my_x = lax.axis_index("x")
        my_y = lax.axis_index("y")
        nbr = (my_x, 1 - my_y)

        barrier_sem = pltpu.get_barrier_semaphore()
        pl.semaphore_signal(
            barrier_sem, inc=1, device_id=nbr,
            device_id_type=pl.DeviceIdType.MESH,
        )

        in_copies = []
        for c in range(C):
            cp = pltpu.make_async_copy(
                x_hbm.at[c], x_vmem.at[c], in_sems.at[c]
            )
            cp.start()
            in_copies.append(cp)

        pl.semaphore_wait(barrier_sem, 1)

        rdmas = []
        for c in range(C):
            in_copies[c].wait()
            xc = x_vmem[c]
            send_pack[c, :] = jnp.sum(xc * xc, axis=1)
            rdma = pltpu.make_async_remote_copy(
                src_ref=send_pack.at[pl.ds(c, 1), :],
                dst_ref=recv_pack.at[pl.ds(c, 1), :],
                send_sem=send_sems.at[c],
                recv_sem=recv_sems.at[c],
                device_id=nbr,
                device_id_type=pl.DeviceIdType.MESH,
            )
            rdma.start()
            rdmas.append(rdma)

        out_copies = []
        for c in range(C):
            rdmas[c].wait_recv()
            total = send_pack[pl.ds(c, 1), :] + recv_pack[pl.ds(c, 1), :]
            inv_rms = lax.rsqrt(total * (1.0 / N_GLOBAL) + EPS)
            inv_col = jnp.reshape(inv_rms, (rows, 1))
            o_vmem[c] = (x_vmem[c] * g_ref[:, :] * inv_col).astype(jnp.bfloat16)
            cp = pltpu.make_async_copy(
                o_vmem.at[c], o_hbm.at[c], out_sems.at[c]
            )
            cp.start()
            out_copies.append(cp)

        for c in range(C):
            rdmas[c].wait_send()
            out_copies[c].wait()

    out3 = pl.pallas_call(
        body,
        out_shape=jax.ShapeDtypeStruct((C, rows, n), jnp.bfloat16),
        in_specs=[
            pl.BlockSpec(memory_space=pl.ANY),
            pl.BlockSpec(memory_space=pltpu.VMEM),
        ],
        out_specs=pl.BlockSpec(memory_space=pl.ANY),
        scratch_shapes=[
            pltpu.VMEM((C, rows, n), jnp.float32),
            pltpu.VMEM((C, rows, n), jnp.bfloat16),
            pltpu.VMEM((C, rows), jnp.float32),
            pltpu.VMEM((C, rows), jnp.float32),
            pltpu.SemaphoreType.DMA((C,)),
            pltpu.SemaphoreType.DMA((C,)),
            pltpu.SemaphoreType.DMA((C,)),
            pltpu.SemaphoreType.DMA((C,)),
        ],
        compiler_params=pltpu.CompilerParams(collective_id=0),
    )(x3, gamma2)
    return out3.reshape(m, n)
